# Initial kernel scaffold; baseline (speedup 1.0000x reference)
import jax
import jax.numpy as jnp
from jax.experimental import pallas as pl


def kernel(im1, im2, C, M1, M2):
    raise NotImplementedError("write your pallas kernel here")


# speedup vs baseline: 1.0494x; 1.0494x over previous
import jax
import jax.numpy as jnp
from jax import lax
from jax.experimental import pallas as pl
from jax.experimental.pallas import tpu as pltpu
from jax.experimental.pallas import tpu_sc as plsc

D = 224
N = 32
R = D * D
CHUNK = 128
NCHUNK = R // CHUNK
NG = CHUNK // 16

_LO = 0.001
_HI = D - 1.001


def _axis_terms(qo, c_chunk, sign):
    q = qo + sign * c_chunk
    qc = jnp.minimum(jnp.maximum(q, _LO), _HI)
    fi = qc.astype(jnp.int32)
    ff = fi.astype(jnp.float32)
    frac_pos = qc > ff
    cf = ff + jnp.where(frac_pos, 1.0, 0.0)
    wf = 1.0 - (qc - ff)
    wc = 1.0 - (cf - qc)
    ca = wf + jnp.where(frac_pos, 0.0, wc)
    cb = jnp.where(frac_pos, wc, 0.0)
    d = q - qc
    return fi, ca, cb, d * d


def _sc_body(t1, t2, aux, out, lossv,
             auxv, iaf, iac, ibf, ibc,
             raA, rbA, caA, cbA, raB, rbB, caB, cbB,
             g1f, g1c, g2f, g2c, outv, accs, sem):
    wid = lax.axis_index("s") * 2 + lax.axis_index("c")
    n = wid
    iot = lax.iota(jnp.int32, 16)
    nR = n * R

    def chunk(g, acc):
        base = g * CHUNK
        pltpu.sync_copy(aux.at[pl.ds((n * NCHUNK + g) * (4 * CHUNK), 4 * CHUNK)],
                        auxv)
        for g2 in range(NG):
            s = g2 * 16
            c0 = auxv[pl.ds(s, 16)]
            c1 = auxv[pl.ds(CHUNK + s, 16)]
            p = base + s + iot
            q0 = lax.div(p, D).astype(jnp.float32)
            q1 = lax.rem(p, D).astype(jnp.float32)
            f0, ra, rb, d0 = _axis_terms(q0, c0, 1.0)
            f1, cca, ccb, d1 = _axis_terms(q1, c1, 1.0)
            idx = nR + f0 * D + f1
            iaf[pl.ds(s, 16)] = idx
            iac[pl.ds(s, 16)] = idx + D
            raA[pl.ds(s, 16)] = ra
            rbA[pl.ds(s, 16)] = rb
            caA[pl.ds(s, 16)] = cca
            cbA[pl.ds(s, 16)] = ccb
            acc = acc + d0 + d1
            f0, ra, rb, d0 = _axis_terms(q0, c0, -1.0)
            f1, cca, ccb, d1 = _axis_terms(q1, c1, -1.0)
            idx = nR + f0 * D + f1
            ibf[pl.ds(s, 16)] = idx
            ibc[pl.ds(s, 16)] = idx + D
            raB[pl.ds(s, 16)] = ra
            rbB[pl.ds(s, 16)] = rb
            caB[pl.ds(s, 16)] = cca
            cbB[pl.ds(s, 16)] = ccb
            acc = acc + d0 + d1
        cp1 = pltpu.async_copy(t1.at[iaf], g1f, sem)
        cp2 = pltpu.async_copy(t1.at[iac], g1c, sem)
        cp3 = pltpu.async_copy(t2.at[ibf], g2f, sem)
        cp4 = pltpu.async_copy(t2.at[ibc], g2c, sem)
        cp1.wait()
        cp2.wait()
        cp3.wait()
        cp4.wait()
        for g2 in range(NG):
            s = g2 * 16
            rows = s + iot
            m1 = auxv[pl.ds(2 * CHUNK + s, 16)]
            m2 = auxv[pl.ds(3 * CHUNK + s, 16)]
            ra1 = raA[pl.ds(s, 16)]
            rb1 = rbA[pl.ds(s, 16)]
            ca1 = caA[pl.ds(s, 16)]
            cb1 = cbA[pl.ds(s, 16)]
            ra2 = raB[pl.ds(s, 16)]
            rb2 = rbB[pl.ds(s, 16)]
            ca2 = caB[pl.ds(s, 16)]
            cb2 = cbB[pl.ds(s, 16)]
            for ch in range(3):
                c_lo = jnp.full((16,), ch, jnp.int32)
                c_hi = jnp.full((16,), ch + 3, jnp.int32)
                v1 = (ra1 * (ca1 * plsc.load_gather(g1f, [rows, c_lo])
                             + cb1 * plsc.load_gather(g1f, [rows, c_hi]))
                      + rb1 * (ca1 * plsc.load_gather(g1c, [rows, c_lo])
                               + cb1 * plsc.load_gather(g1c, [rows, c_hi])))
                v2 = (ra2 * (ca2 * plsc.load_gather(g2f, [rows, c_lo])
                             + cb2 * plsc.load_gather(g2f, [rows, c_hi]))
                      + rb2 * (ca2 * plsc.load_gather(g2c, [rows, c_lo])
                               + cb2 * plsc.load_gather(g2c, [rows, c_hi])))
                outv[pl.ds(ch * CHUNK + s, 16)] = v1 * m1 + v2 * m2
        for ch in range(3):
            pltpu.sync_copy(
                outv.at[pl.ds(ch * CHUNK, CHUNK)],
                out.at[pl.ds((n * 3 + ch) * R + base, CHUNK)])
        return acc

    acc = lax.fori_loop(0, NCHUNK, chunk, jnp.zeros((16,), jnp.float32))
    accs[...] = acc
    pltpu.sync_copy(accs, lossv.at[pl.ds(wid * 16, 16)])


@jax.jit
def _run(t1, t2, aux):
    mesh = plsc.VectorSubcoreMesh(core_axis_name="c", subcore_axis_name="s")
    f = pl.kernel(
        _sc_body,
        out_type=[
            jax.ShapeDtypeStruct((N * 3 * R,), jnp.float32),
            jax.ShapeDtypeStruct((N * 16,), jnp.float32),
        ],
        mesh=mesh,
        compiler_params=pltpu.CompilerParams(
            needs_layout_passes=False, use_tc_tiling_on_sc=False),
        scratch_types=[
            pltpu.VMEM((4 * CHUNK,), jnp.float32),
            pltpu.VMEM((CHUNK,), jnp.int32),
            pltpu.VMEM((CHUNK,), jnp.int32),
            pltpu.VMEM((CHUNK,), jnp.int32),
            pltpu.VMEM((CHUNK,), jnp.int32),
            pltpu.VMEM((CHUNK,), jnp.float32),
            pltpu.VMEM((CHUNK,), jnp.float32),
            pltpu.VMEM((CHUNK,), jnp.float32),
            pltpu.VMEM((CHUNK,), jnp.float32),
            pltpu.VMEM((CHUNK,), jnp.float32),
            pltpu.VMEM((CHUNK,), jnp.float32),
            pltpu.VMEM((CHUNK,), jnp.float32),
            pltpu.VMEM((CHUNK,), jnp.float32),
            pltpu.VMEM((CHUNK, 8), jnp.float32),
            pltpu.VMEM((CHUNK, 8), jnp.float32),
            pltpu.VMEM((CHUNK, 8), jnp.float32),
            pltpu.VMEM((CHUNK, 8), jnp.float32),
            pltpu.VMEM((3 * CHUNK,), jnp.float32),
            pltpu.VMEM((16,), jnp.float32),
            pltpu.SemaphoreType.DMA,
        ],
    )
    return f(t1, t2, aux)


def _pair_table(im):
    cl = jnp.transpose(im, (0, 2, 3, 1))
    nxt = jnp.roll(cl, -1, axis=2)
    t = jnp.concatenate(
        [cl, nxt, jnp.zeros((N, D, D, 2), cl.dtype)], axis=-1)
    return t.reshape(N * R, 8)


def kernel(im1, im2, C, M1, M2):
    t1 = _pair_table(im1)
    t2 = _pair_table(im2)
    aux = jnp.concatenate(
        [C.reshape(N, 2, NCHUNK, CHUNK),
         M1.reshape(N, 1, NCHUNK, CHUNK),
         M2.reshape(N, 1, NCHUNK, CHUNK)], axis=1)
    aux = jnp.transpose(aux, (0, 2, 1, 3)).reshape(N * NCHUNK * 4 * CHUNK)
    out_flat, loss_part = _run(t1, t2, aux)
    out = out_flat.reshape(N, 3, D, D)
    loss = jnp.sum(loss_part) * (0.01 / (N * 2.0 * R * D * D))
    return out, loss


# speedup vs baseline: 1.1448x; 1.0909x over previous
import jax
import jax.numpy as jnp
from jax import lax
from jax.experimental import pallas as pl
from jax.experimental.pallas import tpu as pltpu
from jax.experimental.pallas import tpu_sc as plsc

D = 224
N = 32
R = D * D
CHUNK = 256
NSUB = CHUNK // 128
NCHUNK = R // CHUNK
NG = CHUNK // 16

_LO = 0.001
_HI = D - 1.001


def _axis_terms(qo, c_chunk, sign):
    q = qo + sign * c_chunk
    qc = jnp.minimum(jnp.maximum(q, _LO), _HI)
    fi = qc.astype(jnp.int32)
    ff = fi.astype(jnp.float32)
    frac_pos = qc > ff
    cf = ff + jnp.where(frac_pos, 1.0, 0.0)
    wf = 1.0 - (qc - ff)
    wc = 1.0 - (cf - qc)
    ca = wf + jnp.where(frac_pos, 0.0, wc)
    cb = jnp.where(frac_pos, wc, 0.0)
    d = q - qc
    return fi, ca, cb, d * d


def _sc_body(t1, t2, aux, out, lossv,
             auxv0, auxv1, ia0, ia1, ib0, ib1,
             cf0, cf1, g0, g1, outv0, outv1, accs,
             semg0, semg1, semo0, semo1):
    wid = lax.axis_index("s") * 2 + lax.axis_index("c")
    n = wid
    iot = lax.iota(jnp.int32, 16)
    nR = n * R
    auxs = (auxv0, auxv1)
    ias = (ia0, ia1)
    ibs = (ib0, ib1)
    cfs = (cf0, cf1)
    gs = (g0, g1)
    outvs = (outv0, outv1)
    semgs = (semg0, semg1)
    semos = (semo0, semo1)

    def phase1(g, b, acc):
        auxv, ia, ib, cf = auxs[b], ias[b], ibs[b], cfs[b]
        base = g * CHUNK
        pltpu.sync_copy(aux.at[pl.ds((n * NCHUNK + g) * (4 * CHUNK),
                                     4 * CHUNK)], auxv)
        for g2 in range(NG):
            s = g2 * 16
            j, sj = divmod(s, 128)
            c0 = auxv[pl.ds(s, 16)]
            c1 = auxv[pl.ds(CHUNK + s, 16)]
            p = base + s + iot
            q0 = lax.div(p, D).astype(jnp.float32)
            q1 = lax.rem(p, D).astype(jnp.float32)
            f0, ra, rb, d0 = _axis_terms(q0, c0, 1.0)
            f1, cca, ccb, d1 = _axis_terms(q1, c1, 1.0)
            idx = nR + f0 * D + f1
            ia[j, pl.ds(sj, 16)] = idx
            ia[NSUB + j, pl.ds(sj, 16)] = idx + D
            cf[0, pl.ds(s, 16)] = ra
            cf[1, pl.ds(s, 16)] = rb
            cf[2, pl.ds(s, 16)] = cca
            cf[3, pl.ds(s, 16)] = ccb
            acc = acc + d0 + d1
            f0, ra, rb, d0 = _axis_terms(q0, c0, -1.0)
            f1, cca, ccb, d1 = _axis_terms(q1, c1, -1.0)
            idx = nR + f0 * D + f1
            ib[j, pl.ds(sj, 16)] = idx
            ib[NSUB + j, pl.ds(sj, 16)] = idx + D
            cf[4, pl.ds(s, 16)] = ra
            cf[5, pl.ds(s, 16)] = rb
            cf[6, pl.ds(s, 16)] = cca
            cf[7, pl.ds(s, 16)] = ccb
            acc = acc + d0 + d1
        gb, sg = gs[b], semgs[b]
        for j in range(NSUB):
            pltpu.async_copy(t1.at[ia.at[j]],
                             gb.at[pl.ds(j * 128, 128)], sg)
            pltpu.async_copy(t1.at[ia.at[NSUB + j]],
                             gb.at[pl.ds(CHUNK + j * 128, 128)], sg)
            pltpu.async_copy(t2.at[ib.at[j]],
                             gb.at[pl.ds(2 * CHUNK + j * 128, 128)], sg)
            pltpu.async_copy(t2.at[ib.at[NSUB + j]],
                             gb.at[pl.ds(3 * CHUNK + j * 128, 128)], sg)
        return acc

    def wait_gathers(b):
        gb, sg = gs[b], semgs[b]
        for j in range(NSUB):
            for r in range(4):
                pltpu.make_async_copy(
                    t1.at[ias[b].at[j]],
                    gb.at[pl.ds(r * CHUNK + j * 128, 128)], sg).wait()

    def phase2(g, b):
        auxv, cf, gb, outv = auxs[b], cfs[b], gs[b], outvs[b]
        base = g * CHUNK
        @pl.when(g >= 2)
        def _():
            for ch in range(3):
                pltpu.make_async_copy(
                    outv.at[pl.ds(ch * CHUNK, CHUNK)],
                    out.at[pl.ds(ch * CHUNK, CHUNK)], semos[b]).wait()
        for g2 in range(NG):
            s = g2 * 16
            rows = s + iot
            m1 = auxv[pl.ds(2 * CHUNK + s, 16)]
            m2 = auxv[pl.ds(3 * CHUNK + s, 16)]
            ra1 = cf[0, pl.ds(s, 16)]
            rb1 = cf[1, pl.ds(s, 16)]
            ca1 = cf[2, pl.ds(s, 16)]
            cb1 = cf[3, pl.ds(s, 16)]
            ra2 = cf[4, pl.ds(s, 16)]
            rb2 = cf[5, pl.ds(s, 16)]
            ca2 = cf[6, pl.ds(s, 16)]
            cb2 = cf[7, pl.ds(s, 16)]
            for ch in range(3):
                c_lo = jnp.full((16,), ch, jnp.int32)
                c_hi = jnp.full((16,), ch + 3, jnp.int32)
                r1f = rows
                r1c = rows + CHUNK
                r2f = rows + 2 * CHUNK
                r2c = rows + 3 * CHUNK
                v1 = (ra1 * (ca1 * plsc.load_gather(gb, [r1f, c_lo])
                             + cb1 * plsc.load_gather(gb, [r1f, c_hi]))
                      + rb1 * (ca1 * plsc.load_gather(gb, [r1c, c_lo])
                               + cb1 * plsc.load_gather(gb, [r1c, c_hi])))
                v2 = (ra2 * (ca2 * plsc.load_gather(gb, [r2f, c_lo])
                             + cb2 * plsc.load_gather(gb, [r2f, c_hi]))
                      + rb2 * (ca2 * plsc.load_gather(gb, [r2c, c_lo])
                               + cb2 * plsc.load_gather(gb, [r2c, c_hi])))
                outv[pl.ds(ch * CHUNK + s, 16)] = v1 * m1 + v2 * m2
        for ch in range(3):
            pltpu.async_copy(
                outv.at[pl.ds(ch * CHUNK, CHUNK)],
                out.at[pl.ds((n * 3 + ch) * R + base, CHUNK)], semos[b])

    acc0 = phase1(0, 0, jnp.zeros((16,), jnp.float32))

    def pair(k, acc):
        for b in range(2):
            g = 2 * k + b
            acc = lax.cond(
                g + 1 < NCHUNK,
                lambda a: phase1(g + 1, 1 - b, a),
                lambda a: a,
                acc)
            wait_gathers(b)
            phase2(g, b)
        return acc

    acc = lax.fori_loop(0, NCHUNK // 2, pair, acc0)
    for b in range(2):
        for ch in range(3):
            pltpu.make_async_copy(
                outvs[b].at[pl.ds(ch * CHUNK, CHUNK)],
                out.at[pl.ds(ch * CHUNK, CHUNK)], semos[b]).wait()
    accs[...] = acc
    pltpu.sync_copy(accs, lossv.at[pl.ds(wid * 16, 16)])


@jax.jit
def _run(t1, t2, aux):
    mesh = plsc.VectorSubcoreMesh(core_axis_name="c", subcore_axis_name="s")
    f = pl.kernel(
        _sc_body,
        out_type=[
            jax.ShapeDtypeStruct((N * 3 * R,), jnp.float32),
            jax.ShapeDtypeStruct((N * 16,), jnp.float32),
        ],
        mesh=mesh,
        compiler_params=pltpu.CompilerParams(
            needs_layout_passes=False, use_tc_tiling_on_sc=False),
        scratch_types=[
            pltpu.VMEM((4 * CHUNK,), jnp.float32),
            pltpu.VMEM((4 * CHUNK,), jnp.float32),
            pltpu.VMEM((2 * NSUB, 128), jnp.int32),
            pltpu.VMEM((2 * NSUB, 128), jnp.int32),
            pltpu.VMEM((2 * NSUB, 128), jnp.int32),
            pltpu.VMEM((2 * NSUB, 128), jnp.int32),
            pltpu.VMEM((8, CHUNK), jnp.float32),
            pltpu.VMEM((8, CHUNK), jnp.float32),
            pltpu.VMEM((4 * CHUNK, 8), jnp.float32),
            pltpu.VMEM((4 * CHUNK, 8), jnp.float32),
            pltpu.VMEM((3 * CHUNK,), jnp.float32),
            pltpu.VMEM((3 * CHUNK,), jnp.float32),
            pltpu.VMEM((16,), jnp.float32),
            pltpu.SemaphoreType.DMA,
            pltpu.SemaphoreType.DMA,
            pltpu.SemaphoreType.DMA,
            pltpu.SemaphoreType.DMA,
        ],
    )
    return f(t1, t2, aux)


def _pair_table(im):
    cl = jnp.transpose(im, (0, 2, 3, 1))
    nxt = jnp.roll(cl, -1, axis=2)
    t = jnp.concatenate(
        [cl, nxt, jnp.zeros((N, D, D, 2), cl.dtype)], axis=-1)
    return t.reshape(N * R, 8)


def kernel(im1, im2, C, M1, M2):
    t1 = _pair_table(im1)
    t2 = _pair_table(im2)
    aux = jnp.concatenate(
        [C.reshape(N, 2, NCHUNK, CHUNK),
         M1.reshape(N, 1, NCHUNK, CHUNK),
         M2.reshape(N, 1, NCHUNK, CHUNK)], axis=1)
    aux = jnp.transpose(aux, (0, 2, 1, 3)).reshape(N * NCHUNK * 4 * CHUNK)
    out_flat, loss_part = _run(t1, t2, aux)
    out = out_flat.reshape(N, 3, D, D)
    loss = jnp.sum(loss_part) * (0.01 / (N * 2.0 * R * D * D))
    return out, loss
